# baseline (speedup 1.0000x reference)
import functools

import jax
import jax.numpy as jnp
from jax import lax
from jax.experimental import pallas as pl
from jax.experimental.pallas import tpu as pltpu
from jax.experimental.pallas import tpu_sc as plsc

_NCHUNKS = 2
_BLOCK_B = 2048

_SC_INFO = plsc.get_sparse_core_info()
_NW = _SC_INFO.num_cores * _SC_INFO.num_subcores


@functools.lru_cache(maxsize=None)
def _make_sc_gather(V, D, B, chunk, nchunks):
  bc = B // nchunks
  b_per_w = bc // _NW
  chunk_base = chunk * bc
  mesh = plsc.VectorSubcoreMesh(core_axis_name="c", subcore_axis_name="s")

  @functools.partial(
      pl.kernel,
      mesh=mesh,
      out_type=jax.ShapeDtypeStruct((bc, D), jnp.float32),
      scratch_types=[
          pltpu.VMEM((b_per_w,), jnp.int32),
          pltpu.VMEM((b_per_w, D), jnp.float32),
          pltpu.SemaphoreType.DMA,
      ],
      name=f"sc_embedding_gather_c{chunk}",
  )
  def gather_kernel(table_hbm, idx_hbm, out_hbm, idx_v, rows_v, sem):
    wid = lax.axis_index("s") * _SC_INFO.num_cores + lax.axis_index("c")
    base = wid * b_per_w
    pltpu.sync_copy(idx_hbm.at[pl.ds(chunk_base + base, b_per_w)], idx_v)
    pltpu.async_copy(table_hbm.at[idx_v], rows_v, sem).wait()
    pltpu.sync_copy(rows_v, out_hbm.at[pl.ds(base, b_per_w)])

  return gather_kernel


def _tc_body_first(feat_ref, id_ref, wf_ref, bf_ref, w1a_ref, w1b_ref,
                   b1_ref, w2_ref, b2_ref, out_ref):
  idr = id_ref[...]
  fr = (jnp.dot(feat_ref[...], wf_ref[...], preferred_element_type=jnp.float32)
        + bf_ref[...])
  h = jnp.dot(idr, w1a_ref[...], preferred_element_type=jnp.float32)
  h += jnp.dot(fr, w1b_ref[...], preferred_element_type=jnp.float32)
  h = jnp.maximum(h + b1_ref[...], 0.0)
  g = jnp.dot(h, w2_ref[...], preferred_element_type=jnp.float32) + b2_ref[...]
  gate = jax.nn.sigmoid(g)
  out_ref[...] = gate * idr + (1.0 - gate) * fr


def _tc_body_chained(feat_ref, id_ref, wf_ref, bf_ref, w1a_ref, w1b_ref,
                     b1_ref, w2_ref, b2_ref, prev_ref, out_ref):
  del prev_ref
  _tc_body_first(feat_ref, id_ref, wf_ref, bf_ref, w1a_ref, w1b_ref,
                 b1_ref, w2_ref, b2_ref, out_ref)


def _tc_fused_chunk(features, id_chunk, W_feat, b_feat, W1, b1, W2, b2,
                    prev, chunk, nchunks, block_b):
  B, F = features.shape
  D = id_chunk.shape[1]
  H = W1.shape[1]
  bc = B // nchunks
  nblk = bc // block_b
  base_blk = chunk * nblk
  full = lambda *s: pl.BlockSpec(s, lambda i: (0,) * len(s))
  in_specs = [
      pl.BlockSpec((block_b, F), lambda i: (base_blk + i, 0)),
      pl.BlockSpec((block_b, D), lambda i: (i, 0)),
      full(F, D),
      pl.BlockSpec((1, D), lambda i: (0, 0)),
      pl.BlockSpec((D, H), lambda i: (0, 0)),
      pl.BlockSpec((D, H), lambda i: (1, 0)),
      pl.BlockSpec((1, H), lambda i: (0, 0)),
      full(H, D),
      pl.BlockSpec((1, D), lambda i: (0, 0)),
  ]
  args = [features, id_chunk, W_feat, b_feat, W1, W1, b1, W2, b2]
  kwargs = {}
  if prev is None:
    body = _tc_body_first
  else:
    body = _tc_body_chained
    in_specs.append(pl.BlockSpec(memory_space=pl.ANY))
    args.append(prev)
    kwargs["input_output_aliases"] = {9: 0}
  return pl.pallas_call(
      body,
      grid=(nblk,),
      in_specs=in_specs,
      out_specs=pl.BlockSpec((block_b, D), lambda i: (base_blk + i, 0)),
      out_shape=jax.ShapeDtypeStruct((B, D), jnp.float32),
      **kwargs,
  )(*args)


@jax.jit
def kernel(indices, features, table, W_feat, b_feat, W1, b1, W2, b2):
  V, D = table.shape
  B = indices.shape[0]
  H = W1.shape[1]
  idx = indices.astype(jnp.int32)
  bf = b_feat.reshape(1, D)
  b1r = b1.reshape(1, H)
  b2r = b2.reshape(1, D)
  id_chunks = [
      _make_sc_gather(V, D, B, c, _NCHUNKS)(table, idx)
      for c in range(_NCHUNKS)
  ]
  out = None
  for c in range(_NCHUNKS):
    out = _tc_fused_chunk(features, id_chunks[c], W_feat, bf, W1, b1r, W2,
                          b2r, out, c, _NCHUNKS, _BLOCK_B)
  return out
